# Initial kernel scaffold; baseline (speedup 1.0000x reference)
import jax
import jax.numpy as jnp
from jax.experimental import pallas as pl


def kernel(x, edge_index, W1, b1, W2, b2):
    raise NotImplementedError("write your pallas kernel here")


# speedup vs baseline: 32.0970x; 32.0970x over previous
import functools

import jax
import jax.numpy as jnp
from jax import lax
from jax.experimental import pallas as pl
from jax.experimental.pallas import tpu as pltpu
from jax.experimental.pallas import tpu_sc as plsc

N = 10000
NP = 10240
SLICE = NP // 16
H1 = 16
L = 16

_MESH = dict(core_axis_name="c", subcore_axis_name="s")
_SC_PARAMS = pltpu.CompilerParams(use_tc_tiling_on_sc=False)


def _fill(ref, n, value):
    def body(i, _):
        ref[pl.ds(i * L, L)] = jnp.full((L,), value, jnp.float32)
        return 0
    lax.fori_loop(0, n // L, body, 0)


def _deg_body(ew, w, dst_hbm, deg_hbm, deg_sp, zbuf, ones, idxb):
    cid = lax.axis_index("c")
    sid = lax.axis_index("s")

    @pl.when(cid == 0)
    def _():
        _fill(zbuf, SLICE, 0.0)
        _fill(ones, w, 1.0)
        sl = pl.ds(sid * SLICE, SLICE)
        pltpu.sync_copy(zbuf, deg_sp.at[sl])
        plsc.subcore_barrier()
        base = sid * ew

        def win(i, _):
            pltpu.sync_copy(dst_hbm.at[pl.ds(base + i * w, w)], idxb)
            pltpu.sync_copy(ones, deg_sp.at[idxb], add=True)
            return 0
        lax.fori_loop(0, ew // w, win, 0)
        plsc.subcore_barrier()
        pltpu.sync_copy(deg_sp.at[sl], deg_hbm.at[sl])


def _deg_hist(dst):
    e = dst.shape[0]
    ew = e // 16
    w = 2000
    assert ew % w == 0 and w % 8 == 0
    return pl.kernel(
        functools.partial(_deg_body, ew, w),
        out_type=jax.ShapeDtypeStruct((NP,), jnp.float32),
        mesh=plsc.VectorSubcoreMesh(**_MESH),
        compiler_params=_SC_PARAMS,
        scratch_types=[
            pltpu.VMEM_SHARED((NP,), jnp.float32),
            pltpu.VMEM((SLICE,), jnp.float32),
            pltpu.VMEM((w,), jnp.float32),
            pltpu.VMEM((w,), jnp.int32),
        ],
    )(dst)


def _tc1_body(x_ref, w_ref, deg_ref, g1_ref, dis_ref):
    h = jnp.dot(x_ref[...], w_ref[...], preferred_element_type=jnp.float32)
    d = lax.rsqrt(deg_ref[...] + 1.0)
    dis_ref[...] = d
    g1_ref[...] = h * d


def _tc1(xp, w1, degc):
    blk = 1024
    return pl.pallas_call(
        _tc1_body,
        grid=(NP // blk,),
        in_specs=[
            pl.BlockSpec((blk, xp.shape[1]), lambda i: (i, 0)),
            pl.BlockSpec((xp.shape[1], H1), lambda i: (0, 0)),
            pl.BlockSpec((blk, 1), lambda i: (i, 0)),
        ],
        out_specs=[
            pl.BlockSpec((blk, H1), lambda i: (i, 0)),
            pl.BlockSpec((blk, 1), lambda i: (i, 0)),
        ],
        out_shape=[
            jax.ShapeDtypeStruct((NP, H1), jnp.float32),
            jax.ShapeDtypeStruct((NP, 1), jnp.float32),
        ],
    )(xp, w1, degc)


def _agg1_body(e32, w, g1_hbm, src_hbm, dst_hbm, p0_hbm, p1_hbm,
               acc_sp, cbuf, sidx, didx, rows, sem):
    cid = lax.axis_index("c")
    sid = lax.axis_index("s")
    sl = pl.ds(sid * SLICE, SLICE)

    @pl.when(cid == 0)
    def _():
        pltpu.sync_copy(g1_hbm.at[sl], acc_sp.at[sl])

    @pl.when(cid == 1)
    def _():
        def z(i, _):
            cbuf[i] = jnp.zeros((L,), jnp.float32)
            return 0
        lax.fori_loop(0, SLICE, z, 0)
        pltpu.sync_copy(cbuf, acc_sp.at[sl])

    plsc.subcore_barrier()
    base = (cid * 16 + sid) * e32

    def win(i, _):
        pltpu.sync_copy(src_hbm.at[pl.ds(base + i * w, w)], sidx)
        pltpu.sync_copy(dst_hbm.at[pl.ds(base + i * w, w)], didx)
        pltpu.async_copy(g1_hbm.at[sidx], rows, sem).wait()
        pltpu.sync_copy(rows, acc_sp.at[didx], add=True)
        return 0
    lax.fori_loop(0, e32 // w, win, 0)
    plsc.subcore_barrier()

    @pl.when(cid == 0)
    def _():
        pltpu.sync_copy(acc_sp.at[sl], p0_hbm.at[sl])

    @pl.when(cid == 1)
    def _():
        pltpu.sync_copy(acc_sp.at[sl], p1_hbm.at[sl])


def _agg1(g1, src, dst):
    e = src.shape[0]
    e32 = e // 32
    w = 1000
    assert e32 % w == 0 and w % 8 == 0
    return pl.kernel(
        functools.partial(_agg1_body, e32, w),
        out_type=[
            jax.ShapeDtypeStruct((NP, H1), jnp.float32),
            jax.ShapeDtypeStruct((NP, H1), jnp.float32),
        ],
        mesh=plsc.VectorSubcoreMesh(**_MESH),
        compiler_params=_SC_PARAMS,
        scratch_types=[
            pltpu.VMEM_SHARED((NP, H1), jnp.float32),
            pltpu.VMEM((SLICE, H1), jnp.float32),
            pltpu.VMEM((w,), jnp.int32),
            pltpu.VMEM((w,), jnp.int32),
            pltpu.VMEM((w, H1), jnp.float32),
            pltpu.SemaphoreType.DMA,
        ],
    )(g1, src, dst)


def _tc2_body(pa_ref, pb_ref, dis_ref, b1_ref, w2_ref, g2_ref):
    d = dis_ref[...]
    out1 = (pa_ref[...] + pb_ref[...]) * d + b1_ref[...]
    r = jnp.maximum(out1, 0.0)
    h2 = jnp.dot(r, w2_ref[...], preferred_element_type=jnp.float32)
    g2_ref[...] = h2 * d


def _tc2(pa, pb, dis, b1r, w2):
    blk = 2048
    return pl.pallas_call(
        _tc2_body,
        grid=(NP // blk,),
        in_specs=[
            pl.BlockSpec((blk, H1), lambda i: (i, 0)),
            pl.BlockSpec((blk, H1), lambda i: (i, 0)),
            pl.BlockSpec((blk, 1), lambda i: (i, 0)),
            pl.BlockSpec((1, H1), lambda i: (0, 0)),
            pl.BlockSpec((H1, 1), lambda i: (0, 0)),
        ],
        out_specs=pl.BlockSpec((blk, 1), lambda i: (i, 0)),
        out_shape=jax.ShapeDtypeStruct((NP, 1), jnp.float32),
    )(pa, pb, dis, b1r, w2)


def _agg2_body(ew, w, g2_hbm, src_hbm, dst_hbm, dis_hbm, b2_hbm, out_hbm,
               acc_sp, sidx, didx, vals, abuf, dbuf, b2buf, sem):
    cid = lax.axis_index("c")
    sid = lax.axis_index("s")

    @pl.when(cid == 0)
    def _():
        sl = pl.ds(sid * SLICE, SLICE)
        pltpu.sync_copy(g2_hbm.at[sl], acc_sp.at[sl])
        pltpu.sync_copy(b2_hbm, b2buf)
        plsc.subcore_barrier()
        base = sid * ew

        def win(i, _):
            pltpu.sync_copy(src_hbm.at[pl.ds(base + i * w, w)], sidx)
            pltpu.sync_copy(dst_hbm.at[pl.ds(base + i * w, w)], didx)
            pltpu.async_copy(g2_hbm.at[sidx], vals, sem).wait()
            pltpu.sync_copy(vals, acc_sp.at[didx], add=True)
            return 0
        lax.fori_loop(0, ew // w, win, 0)
        plsc.subcore_barrier()

        pltpu.sync_copy(acc_sp.at[sl], abuf)
        pltpu.sync_copy(dis_hbm.at[sl], dbuf)
        b2v = b2buf[...]

        def fin(i, _):
            s = pl.ds(i * L, L)
            abuf[s] = abuf[s] * dbuf[s] + b2v
            return 0
        lax.fori_loop(0, SLICE // L, fin, 0)
        pltpu.sync_copy(abuf, out_hbm.at[sl])


def _agg2(g2, src, dst, dis, b2t):
    e = src.shape[0]
    ew = e // 16
    w = 2000
    assert ew % w == 0 and w % 8 == 0
    return pl.kernel(
        functools.partial(_agg2_body, ew, w),
        out_type=jax.ShapeDtypeStruct((NP,), jnp.float32),
        mesh=plsc.VectorSubcoreMesh(**_MESH),
        compiler_params=_SC_PARAMS,
        scratch_types=[
            pltpu.VMEM_SHARED((NP,), jnp.float32),
            pltpu.VMEM((w,), jnp.int32),
            pltpu.VMEM((w,), jnp.int32),
            pltpu.VMEM((w,), jnp.float32),
            pltpu.VMEM((SLICE,), jnp.float32),
            pltpu.VMEM((SLICE,), jnp.float32),
            pltpu.VMEM((L,), jnp.float32),
            pltpu.SemaphoreType.DMA,
        ],
    )(g2, src, dst, dis, b2t)


def kernel(x, edge_index, W1, b1, W2, b2):
    src = edge_index[0]
    dst = edge_index[1]
    xp = jnp.pad(x, ((0, NP - N), (0, 0)))
    deg = _deg_hist(dst)
    g1, dis = _tc1(xp, W1, deg.reshape(NP, 1))
    p0, p1 = _agg1(g1, src, dst)
    g2 = _tc2(p0, p1, dis, b1.reshape(1, H1), W2)
    b2t = jnp.tile(b2, L)
    outp = _agg2(g2.reshape(NP), src, dst, dis.reshape(NP), b2t)
    return outp[:N].reshape(N, 1)


# speedup vs baseline: 33.1827x; 1.0338x over previous
import functools

import jax
import jax.numpy as jnp
from jax import lax
from jax.experimental import pallas as pl
from jax.experimental.pallas import tpu as pltpu
from jax.experimental.pallas import tpu_sc as plsc

N = 10000
NP = 10240
SLICE = NP // 16
H1 = 16
L = 16

_MESH = dict(core_axis_name="c", subcore_axis_name="s")
_SC_PARAMS = pltpu.CompilerParams(use_tc_tiling_on_sc=False)


def _fill(ref, n, value):
    def body(i, _):
        ref[pl.ds(i * L, L)] = jnp.full((L,), value, jnp.float32)
        return 0
    lax.fori_loop(0, n // L, body, 0)


def _deg_body(ew, w, dst_hbm, deg_hbm, deg_sp, zbuf, ones, idxb):
    cid = lax.axis_index("c")
    sid = lax.axis_index("s")

    @pl.when(cid == 0)
    def _():
        _fill(zbuf, SLICE, 0.0)
        _fill(ones, w, 1.0)
        sl = pl.ds(sid * SLICE, SLICE)
        pltpu.sync_copy(zbuf, deg_sp.at[sl])
        plsc.subcore_barrier()
        base = sid * ew

        def win(i, _):
            pltpu.sync_copy(dst_hbm.at[pl.ds(base + i * w, w)], idxb)
            pltpu.sync_copy(ones, deg_sp.at[idxb], add=True)
            return 0
        lax.fori_loop(0, ew // w, win, 0)
        plsc.subcore_barrier()
        pltpu.sync_copy(deg_sp.at[sl], deg_hbm.at[sl])


def _deg_hist(dst):
    e = dst.shape[0]
    ew = e // 16
    w = 2000
    assert ew % w == 0 and w % 8 == 0
    return pl.kernel(
        functools.partial(_deg_body, ew, w),
        out_type=jax.ShapeDtypeStruct((NP,), jnp.float32),
        mesh=plsc.VectorSubcoreMesh(**_MESH),
        compiler_params=_SC_PARAMS,
        scratch_types=[
            pltpu.VMEM_SHARED((NP,), jnp.float32),
            pltpu.VMEM((SLICE,), jnp.float32),
            pltpu.VMEM((w,), jnp.float32),
            pltpu.VMEM((w,), jnp.int32),
        ],
    )(dst)


def _tc1_body(x_ref, w_ref, deg_ref, g1_ref, dis_ref):
    h = jnp.dot(x_ref[...], w_ref[...], preferred_element_type=jnp.float32)
    d = lax.rsqrt(deg_ref[...] + 1.0)
    dis_ref[...] = d
    g1_ref[...] = h * d


def _tc1(xp, w1, degc):
    blk = 1024
    return pl.pallas_call(
        _tc1_body,
        grid=(NP // blk,),
        in_specs=[
            pl.BlockSpec((blk, xp.shape[1]), lambda i: (i, 0)),
            pl.BlockSpec((xp.shape[1], H1), lambda i: (0, 0)),
            pl.BlockSpec((blk, 1), lambda i: (i, 0)),
        ],
        out_specs=[
            pl.BlockSpec((blk, H1), lambda i: (i, 0)),
            pl.BlockSpec((blk, 1), lambda i: (i, 0)),
        ],
        out_shape=[
            jax.ShapeDtypeStruct((NP, H1), jnp.float32),
            jax.ShapeDtypeStruct((NP, 1), jnp.float32),
        ],
    )(xp, w1, degc)


def _agg1_body(e32, w, nw, g1_hbm, src_hbm, dst_hbm, p0_hbm, p1_hbm,
               acc_sp, cbuf, sidx0, sidx1, didx0, didx1, rows0, rows1,
               gsem, ssem0, ssem1):
    cid = lax.axis_index("c")
    sid = lax.axis_index("s")
    sl = pl.ds(sid * SLICE, SLICE)

    @pl.when(cid == 0)
    def _():
        pltpu.sync_copy(g1_hbm.at[sl], acc_sp.at[sl])

    @pl.when(cid == 1)
    def _():
        def z(i, _):
            cbuf[i] = jnp.zeros((L,), jnp.float32)
            return 0
        lax.fori_loop(0, SLICE, z, 0)
        pltpu.sync_copy(cbuf, acc_sp.at[sl])

    plsc.subcore_barrier()
    base = (cid * 16 + sid) * e32

    descs = [None, None]
    sidx = [sidx0, sidx1]
    didx = [didx0, didx1]
    rows = [rows0, rows1]
    ssem = [ssem0, ssem1]
    for i in range(nw):
        p = i & 1
        if descs[p] is not None:
            descs[p].wait()
        pltpu.sync_copy(src_hbm.at[pl.ds(base + i * w, w)], sidx[p])
        pltpu.sync_copy(dst_hbm.at[pl.ds(base + i * w, w)], didx[p])
        pltpu.async_copy(g1_hbm.at[sidx[p]], rows[p], gsem).wait()
        descs[p] = pltpu.async_copy(rows[p], acc_sp.at[didx[p]], ssem[p],
                                    add=True)
    for d in descs:
        if d is not None:
            d.wait()
    plsc.subcore_barrier()

    @pl.when(cid == 0)
    def _():
        pltpu.sync_copy(acc_sp.at[sl], p0_hbm.at[sl])

    @pl.when(cid == 1)
    def _():
        pltpu.sync_copy(acc_sp.at[sl], p1_hbm.at[sl])


def _agg1(g1, src, dst):
    e = src.shape[0]
    e32 = e // 32
    w = 1000
    nw = e32 // w
    assert e32 % w == 0 and w % 8 == 0
    return pl.kernel(
        functools.partial(_agg1_body, e32, w, nw),
        out_type=[
            jax.ShapeDtypeStruct((NP, H1), jnp.float32),
            jax.ShapeDtypeStruct((NP, H1), jnp.float32),
        ],
        mesh=plsc.VectorSubcoreMesh(**_MESH),
        compiler_params=_SC_PARAMS,
        scratch_types=[
            pltpu.VMEM_SHARED((NP, H1), jnp.float32),
            pltpu.VMEM((SLICE, H1), jnp.float32),
            pltpu.VMEM((w,), jnp.int32),
            pltpu.VMEM((w,), jnp.int32),
            pltpu.VMEM((w,), jnp.int32),
            pltpu.VMEM((w,), jnp.int32),
            pltpu.VMEM((w, H1), jnp.float32),
            pltpu.VMEM((w, H1), jnp.float32),
            pltpu.SemaphoreType.DMA,
            pltpu.SemaphoreType.DMA,
            pltpu.SemaphoreType.DMA,
        ],
    )(g1, src, dst)


def _tc2_body(pa_ref, pb_ref, dis_ref, b1_ref, w2_ref, g2_ref):
    d = dis_ref[...]
    out1 = (pa_ref[...] + pb_ref[...]) * d + b1_ref[...]
    r = jnp.maximum(out1, 0.0)
    h2 = jnp.dot(r, w2_ref[...], preferred_element_type=jnp.float32)
    g2_ref[...] = h2 * d


def _tc2(pa, pb, dis, b1r, w2):
    blk = 2048
    return pl.pallas_call(
        _tc2_body,
        grid=(NP // blk,),
        in_specs=[
            pl.BlockSpec((blk, H1), lambda i: (i, 0)),
            pl.BlockSpec((blk, H1), lambda i: (i, 0)),
            pl.BlockSpec((blk, 1), lambda i: (i, 0)),
            pl.BlockSpec((1, H1), lambda i: (0, 0)),
            pl.BlockSpec((H1, 1), lambda i: (0, 0)),
        ],
        out_specs=pl.BlockSpec((blk, 1), lambda i: (i, 0)),
        out_shape=jax.ShapeDtypeStruct((NP, 1), jnp.float32),
    )(pa, pb, dis, b1r, w2)


def _agg2_body(ew, w, nw, g2_hbm, src_hbm, dst_hbm, dis_hbm, b2_hbm, out_hbm,
               acc_sp, sidx0, sidx1, didx0, didx1, vals0, vals1, abuf, dbuf,
               b2buf, gsem, ssem0, ssem1):
    cid = lax.axis_index("c")
    sid = lax.axis_index("s")

    @pl.when(cid == 0)
    def _():
        sl = pl.ds(sid * SLICE, SLICE)
        pltpu.sync_copy(g2_hbm.at[sl], acc_sp.at[sl])
        pltpu.sync_copy(b2_hbm, b2buf)
        plsc.subcore_barrier()
        base = sid * ew

        descs = [None, None]
        sidx = [sidx0, sidx1]
        didx = [didx0, didx1]
        vals = [vals0, vals1]
        ssem = [ssem0, ssem1]
        for i in range(nw):
            p = i & 1
            if descs[p] is not None:
                descs[p].wait()
            pltpu.sync_copy(src_hbm.at[pl.ds(base + i * w, w)], sidx[p])
            pltpu.sync_copy(dst_hbm.at[pl.ds(base + i * w, w)], didx[p])
            pltpu.async_copy(g2_hbm.at[sidx[p]], vals[p], gsem).wait()
            descs[p] = pltpu.async_copy(vals[p], acc_sp.at[didx[p]],
                                        ssem[p], add=True)
        for d in descs:
            if d is not None:
                d.wait()
        plsc.subcore_barrier()

        pltpu.sync_copy(acc_sp.at[sl], abuf)
        pltpu.sync_copy(dis_hbm.at[sl], dbuf)
        b2v = b2buf[...]

        def fin(i, _):
            s = pl.ds(i * L, L)
            abuf[s] = abuf[s] * dbuf[s] + b2v
            return 0
        lax.fori_loop(0, SLICE // L, fin, 0)
        pltpu.sync_copy(abuf, out_hbm.at[sl])


def _agg2(g2, src, dst, dis, b2t):
    e = src.shape[0]
    ew = e // 16
    w = 2000
    nw = ew // w
    assert ew % w == 0 and w % 8 == 0
    return pl.kernel(
        functools.partial(_agg2_body, ew, w, nw),
        out_type=jax.ShapeDtypeStruct((NP,), jnp.float32),
        mesh=plsc.VectorSubcoreMesh(**_MESH),
        compiler_params=_SC_PARAMS,
        scratch_types=[
            pltpu.VMEM_SHARED((NP,), jnp.float32),
            pltpu.VMEM((w,), jnp.int32),
            pltpu.VMEM((w,), jnp.int32),
            pltpu.VMEM((w,), jnp.int32),
            pltpu.VMEM((w,), jnp.int32),
            pltpu.VMEM((w,), jnp.float32),
            pltpu.VMEM((w,), jnp.float32),
            pltpu.VMEM((SLICE,), jnp.float32),
            pltpu.VMEM((SLICE,), jnp.float32),
            pltpu.VMEM((L,), jnp.float32),
            pltpu.SemaphoreType.DMA,
            pltpu.SemaphoreType.DMA,
            pltpu.SemaphoreType.DMA,
        ],
    )(g2, src, dst, dis, b2t)


def kernel(x, edge_index, W1, b1, W2, b2):
    src = edge_index[0]
    dst = edge_index[1]
    xp = jnp.pad(x, ((0, NP - N), (0, 0)))
    deg = _deg_hist(dst)
    g1, dis = _tc1(xp, W1, deg.reshape(NP, 1))
    p0, p1 = _agg1(g1, src, dst)
    g2 = _tc2(p0, p1, dis, b1.reshape(1, H1), W2)
    b2t = jnp.tile(b2, L)
    outp = _agg2(g2.reshape(NP), src, dst, dis.reshape(NP), b2t)
    return outp[:N].reshape(N, 1)


# speedup vs baseline: 38.1259x; 1.1490x over previous
import functools

import jax
import jax.numpy as jnp
from jax import lax
from jax.experimental import pallas as pl
from jax.experimental.pallas import tpu as pltpu
from jax.experimental.pallas import tpu_sc as plsc

N = 10000
NP = 10240
SLICE = NP // 16
H1 = 16
L = 16

_MESH = dict(core_axis_name="c", subcore_axis_name="s")
_SC_PARAMS = pltpu.CompilerParams(use_tc_tiling_on_sc=False)


def _fill(ref, n, value):
    def body(i, _):
        ref[pl.ds(i * L, L)] = jnp.full((L,), value, jnp.float32)
        return 0
    lax.fori_loop(0, n // L, body, 0)


def _deg_body(ew, w, dst_hbm, deg_hbm, deg_sp, zbuf, ones, idxb):
    cid = lax.axis_index("c")
    sid = lax.axis_index("s")

    @pl.when(cid == 0)
    def _():
        _fill(zbuf, SLICE, 0.0)
        _fill(ones, w, 1.0)
        sl = pl.ds(sid * SLICE, SLICE)
        pltpu.sync_copy(zbuf, deg_sp.at[sl])
        plsc.subcore_barrier()
        base = sid * ew

        def win(i, _):
            pltpu.sync_copy(dst_hbm.at[pl.ds(base + i * w, w)], idxb)
            pltpu.sync_copy(ones, deg_sp.at[idxb], add=True)
            return 0
        lax.fori_loop(0, ew // w, win, 0)
        plsc.subcore_barrier()
        pltpu.sync_copy(deg_sp.at[sl], deg_hbm.at[sl])


def _deg_hist(dst):
    e = dst.shape[0]
    ew = e // 16
    w = 2000
    assert ew % w == 0 and w % 8 == 0
    return pl.kernel(
        functools.partial(_deg_body, ew, w),
        out_type=jax.ShapeDtypeStruct((NP,), jnp.float32),
        mesh=plsc.VectorSubcoreMesh(**_MESH),
        compiler_params=_SC_PARAMS,
        scratch_types=[
            pltpu.VMEM_SHARED((NP,), jnp.float32),
            pltpu.VMEM((SLICE,), jnp.float32),
            pltpu.VMEM((w,), jnp.float32),
            pltpu.VMEM((w,), jnp.int32),
        ],
    )(dst)


def _tc1_body(x_ref, w_ref, deg_ref, g1_ref, dis_ref):
    h = jnp.dot(x_ref[...], w_ref[...], preferred_element_type=jnp.float32)
    d = lax.rsqrt(deg_ref[...] + 1.0)
    dis_ref[...] = d
    g1_ref[...] = h * d


def _tc1(xp, w1, degc):
    blk = 1024
    return pl.pallas_call(
        _tc1_body,
        grid=(NP // blk,),
        in_specs=[
            pl.BlockSpec((blk, xp.shape[1]), lambda i: (i, 0)),
            pl.BlockSpec((xp.shape[1], H1), lambda i: (0, 0)),
            pl.BlockSpec((blk, 1), lambda i: (i, 0)),
        ],
        out_specs=[
            pl.BlockSpec((blk, H1), lambda i: (i, 0)),
            pl.BlockSpec((blk, 1), lambda i: (i, 0)),
        ],
        out_shape=[
            jax.ShapeDtypeStruct((NP, H1), jnp.float32),
            jax.ShapeDtypeStruct((NP, 1), jnp.float32),
        ],
    )(xp, w1, degc)


def _agg1_body(e32, w, nw, g1_hbm, src_hbm, dst_hbm, p0_hbm, p1_hbm,
               acc_sp, cbuf, sidx0, sidx1, didx0, didx1, rows0, rows1,
               gsem, ssem0, ssem1):
    cid = lax.axis_index("c")
    sid = lax.axis_index("s")
    sl = pl.ds(sid * SLICE, SLICE)

    @pl.when(cid == 0)
    def _():
        pltpu.sync_copy(g1_hbm.at[sl], acc_sp.at[sl])

    @pl.when(cid == 1)
    def _():
        def z(i, _):
            cbuf[i] = jnp.zeros((L,), jnp.float32)
            return 0
        lax.fori_loop(0, SLICE, z, 0)
        pltpu.sync_copy(cbuf, acc_sp.at[sl])

    plsc.subcore_barrier()
    base = (cid * 16 + sid) * e32

    descs = [None, None]
    sidx = [sidx0, sidx1]
    didx = [didx0, didx1]
    rows = [rows0, rows1]
    ssem = [ssem0, ssem1]
    for i in range(nw):
        p = i & 1
        if descs[p] is not None:
            descs[p].wait()
        pltpu.sync_copy(src_hbm.at[pl.ds(base + i * w, w)], sidx[p])
        pltpu.sync_copy(dst_hbm.at[pl.ds(base + i * w, w)], didx[p])
        pltpu.async_copy(g1_hbm.at[sidx[p]], rows[p], gsem).wait()
        descs[p] = pltpu.async_copy(rows[p], acc_sp.at[didx[p]], ssem[p],
                                    add=True)
    for d in descs:
        if d is not None:
            d.wait()
    plsc.subcore_barrier()

    @pl.when(cid == 0)
    def _():
        pltpu.sync_copy(acc_sp.at[sl], p0_hbm.at[sl])

    @pl.when(cid == 1)
    def _():
        pltpu.sync_copy(acc_sp.at[sl], p1_hbm.at[sl])


def _agg1(g1, src, dst):
    e = src.shape[0]
    e32 = e // 32
    w = 1000
    nw = e32 // w
    assert e32 % w == 0 and w % 8 == 0
    return pl.kernel(
        functools.partial(_agg1_body, e32, w, nw),
        out_type=[
            jax.ShapeDtypeStruct((NP, H1), jnp.float32),
            jax.ShapeDtypeStruct((NP, H1), jnp.float32),
        ],
        mesh=plsc.VectorSubcoreMesh(**_MESH),
        compiler_params=_SC_PARAMS,
        scratch_types=[
            pltpu.VMEM_SHARED((NP, H1), jnp.float32),
            pltpu.VMEM((SLICE, H1), jnp.float32),
            pltpu.VMEM((w,), jnp.int32),
            pltpu.VMEM((w,), jnp.int32),
            pltpu.VMEM((w,), jnp.int32),
            pltpu.VMEM((w,), jnp.int32),
            pltpu.VMEM((w, H1), jnp.float32),
            pltpu.VMEM((w, H1), jnp.float32),
            pltpu.SemaphoreType.DMA,
            pltpu.SemaphoreType.DMA,
            pltpu.SemaphoreType.DMA,
        ],
    )(g1, src, dst)


def _tc2_body(pa_ref, pb_ref, dis_ref, b1_ref, w2_ref, g2_ref):
    d = dis_ref[...]
    out1 = (pa_ref[...] + pb_ref[...]) * d + b1_ref[...]
    r = jnp.maximum(out1, 0.0)
    h2 = jnp.dot(r, w2_ref[...], preferred_element_type=jnp.float32)
    g2_ref[...] = h2 * d


def _tc2(pa, pb, dis, b1r, w2):
    blk = 2048
    return pl.pallas_call(
        _tc2_body,
        grid=(NP // blk,),
        in_specs=[
            pl.BlockSpec((blk, H1), lambda i: (i, 0)),
            pl.BlockSpec((blk, H1), lambda i: (i, 0)),
            pl.BlockSpec((blk, 1), lambda i: (i, 0)),
            pl.BlockSpec((1, H1), lambda i: (0, 0)),
            pl.BlockSpec((H1, 1), lambda i: (0, 0)),
        ],
        out_specs=pl.BlockSpec((blk, 1), lambda i: (i, 0)),
        out_shape=jax.ShapeDtypeStruct((NP, 1), jnp.float32),
    )(pa, pb, dis, b1r, w2)


def _agg2_body(ew, w, nw, g2_hbm, src_hbm, dst_hbm, dis_hbm, b2_hbm, out_hbm,
               acc_sp, g2t, sidx, didx0, didx1, vals0, vals1, abuf, dbuf,
               b2buf, ssem0, ssem1):
    cid = lax.axis_index("c")
    sid = lax.axis_index("s")

    @pl.when(cid == 0)
    def _():
        sl = pl.ds(sid * SLICE, SLICE)
        pltpu.sync_copy(g2_hbm.at[sl], acc_sp.at[sl])
        pltpu.sync_copy(g2_hbm, g2t)
        pltpu.sync_copy(b2_hbm, b2buf)
        plsc.subcore_barrier()
        base = sid * ew

        descs = [None, None]
        didx = [didx0, didx1]
        vals = [vals0, vals1]
        ssem = [ssem0, ssem1]
        for i in range(nw):
            p = i & 1
            vp = vals[p]
            if descs[p] is not None:
                descs[p].wait()
            pltpu.sync_copy(src_hbm.at[pl.ds(base + i * w, w)], sidx)
            pltpu.sync_copy(dst_hbm.at[pl.ds(base + i * w, w)], didx[p])

            def gat(j, _):
                s = pl.ds(j * L, L)
                vp[s] = plsc.load_gather(g2t, [sidx[s]])
                return 0
            lax.fori_loop(0, w // L, gat, 0)
            descs[p] = pltpu.async_copy(vp, acc_sp.at[didx[p]],
                                        ssem[p], add=True)
        for d in descs:
            if d is not None:
                d.wait()
        plsc.subcore_barrier()

        pltpu.sync_copy(acc_sp.at[sl], abuf)
        pltpu.sync_copy(dis_hbm.at[sl], dbuf)
        b2v = b2buf[...]

        def fin(i, _):
            s = pl.ds(i * L, L)
            abuf[s] = abuf[s] * dbuf[s] + b2v
            return 0
        lax.fori_loop(0, SLICE // L, fin, 0)
        pltpu.sync_copy(abuf, out_hbm.at[sl])


def _agg2(g2, src, dst, dis, b2t):
    e = src.shape[0]
    ew = e // 16
    w = 2000
    nw = ew // w
    assert ew % w == 0 and w % 8 == 0
    return pl.kernel(
        functools.partial(_agg2_body, ew, w, nw),
        out_type=jax.ShapeDtypeStruct((NP,), jnp.float32),
        mesh=plsc.VectorSubcoreMesh(**_MESH),
        compiler_params=pltpu.CompilerParams(use_tc_tiling_on_sc=False,
                                             needs_layout_passes=False),
        scratch_types=[
            pltpu.VMEM_SHARED((NP,), jnp.float32),
            pltpu.VMEM((NP,), jnp.float32),
            pltpu.VMEM((w,), jnp.int32),
            pltpu.VMEM((w,), jnp.int32),
            pltpu.VMEM((w,), jnp.int32),
            pltpu.VMEM((w,), jnp.float32),
            pltpu.VMEM((w,), jnp.float32),
            pltpu.VMEM((SLICE,), jnp.float32),
            pltpu.VMEM((SLICE,), jnp.float32),
            pltpu.VMEM((L,), jnp.float32),
            pltpu.SemaphoreType.DMA,
            pltpu.SemaphoreType.DMA,
        ],
    )(g2, src, dst, dis, b2t)


def kernel(x, edge_index, W1, b1, W2, b2):
    src = edge_index[0]
    dst = edge_index[1]
    xp = jnp.pad(x, ((0, NP - N), (0, 0)))
    deg = _deg_hist(dst)
    g1, dis = _tc1(xp, W1, deg.reshape(NP, 1))
    p0, p1 = _agg1(g1, src, dst)
    g2 = _tc2(p0, p1, dis, b1.reshape(1, H1), W2)
    b2t = jnp.tile(b2, L)
    outp = _agg2(g2.reshape(NP), src, dst, dis.reshape(NP), b2t)
    return outp[:N].reshape(N, 1)


# speedup vs baseline: 41.1327x; 1.0789x over previous
import functools

import jax
import jax.numpy as jnp
from jax import lax
from jax.experimental import pallas as pl
from jax.experimental.pallas import tpu as pltpu
from jax.experimental.pallas import tpu_sc as plsc

N = 10000
NP = 10240
SLICE = NP // 16
H1 = 16
L = 16
ZR = 40

_MESH = dict(core_axis_name="c", subcore_axis_name="s")
_SC_PARAMS = pltpu.CompilerParams(use_tc_tiling_on_sc=False)
_SC_PARAMS_NOLAYOUT = pltpu.CompilerParams(use_tc_tiling_on_sc=False,
                                           needs_layout_passes=False)


def _fill(ref, n, value):
    def body(i, _):
        ref[pl.ds(i * L, L)] = jnp.full((L,), value, jnp.float32)
        return 0
    lax.fori_loop(0, n // L, body, 0)


def _deg_body(ew, dst_hbm, deg_hbm, deg_sp, zbuf, ones, idxb):
    cid = lax.axis_index("c")
    sid = lax.axis_index("s")

    @pl.when(cid == 0)
    def _():
        _fill(zbuf, SLICE, 0.0)
        _fill(ones, ew, 1.0)
        sl = pl.ds(sid * SLICE, SLICE)
        pltpu.sync_copy(zbuf, deg_sp.at[sl])
        pltpu.sync_copy(dst_hbm.at[pl.ds(sid * ew, ew)], idxb)
        plsc.subcore_barrier()
        pltpu.sync_copy(ones, deg_sp.at[idxb], add=True)
        plsc.subcore_barrier()
        pltpu.sync_copy(deg_sp.at[sl], deg_hbm.at[sl])


def _deg_hist(dst):
    e = dst.shape[0]
    ew = e // 16
    assert ew % 8 == 0
    return pl.kernel(
        functools.partial(_deg_body, ew),
        out_type=jax.ShapeDtypeStruct((NP,), jnp.float32),
        mesh=plsc.VectorSubcoreMesh(**_MESH),
        compiler_params=_SC_PARAMS,
        scratch_types=[
            pltpu.VMEM_SHARED((NP,), jnp.float32),
            pltpu.VMEM((SLICE,), jnp.float32),
            pltpu.VMEM((ew,), jnp.float32),
            pltpu.VMEM((ew,), jnp.int32),
        ],
    )(dst)


def _tc1_body(x_ref, w_ref, deg_ref, g1_ref, dis_ref):
    h = jnp.dot(x_ref[...], w_ref[...], preferred_element_type=jnp.float32)
    d = lax.rsqrt(deg_ref[...] + 1.0)
    dis_ref[...] = d
    g1_ref[...] = h * d


def _tc1(xp, w1, degc):
    blk = 1024
    return pl.pallas_call(
        _tc1_body,
        grid=(NP // blk,),
        in_specs=[
            pl.BlockSpec((blk, xp.shape[1]), lambda i: (i, 0)),
            pl.BlockSpec((xp.shape[1], H1), lambda i: (0, 0)),
            pl.BlockSpec((blk, 1), lambda i: (i, 0)),
        ],
        out_specs=[
            pl.BlockSpec((blk, H1), lambda i: (i, 0)),
            pl.BlockSpec((blk, 1), lambda i: (i, 0)),
        ],
        out_shape=[
            jax.ShapeDtypeStruct((NP, H1), jnp.float32),
            jax.ShapeDtypeStruct((NP, 1), jnp.float32),
        ],
    )(xp, w1, degc)


def _agg1_body(e32, g1_hbm, src_hbm, dst_hbm, p0_hbm, p1_hbm,
               acc_sp, zbuf, sidx, didx, rows, gsem):
    cid = lax.axis_index("c")
    sid = lax.axis_index("s")
    sl = pl.ds(sid * SLICE, SLICE)

    def z(i, _):
        zbuf[i] = jnp.zeros((L,), jnp.float32)
        return 0
    lax.fori_loop(0, ZR, z, 0)
    for k in range(SLICE // ZR):
        pltpu.sync_copy(zbuf, acc_sp.at[pl.ds(sid * SLICE + k * ZR, ZR)])

    base = (cid * 16 + sid) * e32
    pltpu.sync_copy(src_hbm.at[pl.ds(base, e32)], sidx)
    pltpu.sync_copy(dst_hbm.at[pl.ds(base, e32)], didx)
    plsc.subcore_barrier()
    pltpu.async_copy(g1_hbm.at[sidx], rows, gsem).wait()
    pltpu.sync_copy(rows, acc_sp.at[didx], add=True)
    plsc.subcore_barrier()

    @pl.when(cid == 0)
    def _():
        pltpu.sync_copy(acc_sp.at[sl], p0_hbm.at[sl])

    @pl.when(cid == 1)
    def _():
        pltpu.sync_copy(acc_sp.at[sl], p1_hbm.at[sl])


def _agg1(g1, src, dst):
    e = src.shape[0]
    e32 = e // 32
    assert e32 % 8 == 0 and SLICE % ZR == 0
    return pl.kernel(
        functools.partial(_agg1_body, e32),
        out_type=[
            jax.ShapeDtypeStruct((NP, H1), jnp.float32),
            jax.ShapeDtypeStruct((NP, H1), jnp.float32),
        ],
        mesh=plsc.VectorSubcoreMesh(**_MESH),
        compiler_params=_SC_PARAMS,
        scratch_types=[
            pltpu.VMEM_SHARED((NP, H1), jnp.float32),
            pltpu.VMEM((ZR, H1), jnp.float32),
            pltpu.VMEM((e32,), jnp.int32),
            pltpu.VMEM((e32,), jnp.int32),
            pltpu.VMEM((e32, H1), jnp.float32),
            pltpu.SemaphoreType.DMA,
        ],
    )(g1, src, dst)


def _tc2_body(pa_ref, pb_ref, g1_ref, dis_ref, b1_ref, w2_ref, g2_ref):
    d = dis_ref[...]
    out1 = (pa_ref[...] + pb_ref[...] + g1_ref[...]) * d + b1_ref[...]
    r = jnp.maximum(out1, 0.0)
    h2 = jnp.dot(r, w2_ref[...], preferred_element_type=jnp.float32)
    g2_ref[...] = h2 * d


def _tc2(pa, pb, g1, dis, b1r, w2):
    blk = 2048
    return pl.pallas_call(
        _tc2_body,
        grid=(NP // blk,),
        in_specs=[
            pl.BlockSpec((blk, H1), lambda i: (i, 0)),
            pl.BlockSpec((blk, H1), lambda i: (i, 0)),
            pl.BlockSpec((blk, H1), lambda i: (i, 0)),
            pl.BlockSpec((blk, 1), lambda i: (i, 0)),
            pl.BlockSpec((1, H1), lambda i: (0, 0)),
            pl.BlockSpec((H1, 1), lambda i: (0, 0)),
        ],
        out_specs=pl.BlockSpec((blk, 1), lambda i: (i, 0)),
        out_shape=jax.ShapeDtypeStruct((NP, 1), jnp.float32),
    )(pa, pb, g1, dis, b1r, w2)


def _agg2_body(ew, g2_hbm, src_hbm, dst_hbm, dis_hbm, b2_hbm, out_hbm,
               acc_sp, g2t, sidx, didx, vals, zbuf, gbuf, dbuf, b2buf):
    cid = lax.axis_index("c")
    sid = lax.axis_index("s")

    @pl.when(cid == 0)
    def _():
        sl = pl.ds(sid * SLICE, SLICE)
        _fill(zbuf, SLICE, 0.0)
        pltpu.sync_copy(zbuf, acc_sp.at[sl])
        pltpu.sync_copy(g2_hbm, g2t)
        pltpu.sync_copy(b2_hbm, b2buf)
        base = sid * ew
        pltpu.sync_copy(src_hbm.at[pl.ds(base, ew)], sidx)
        pltpu.sync_copy(dst_hbm.at[pl.ds(base, ew)], didx)

        def gat(j, _):
            s = pl.ds(j * L, L)
            vals[s] = plsc.load_gather(g2t, [sidx[s]])
            return 0
        lax.fori_loop(0, ew // L, gat, 0)
        plsc.subcore_barrier()
        pltpu.sync_copy(vals, acc_sp.at[didx], add=True)
        plsc.subcore_barrier()

        pltpu.sync_copy(acc_sp.at[sl], gbuf)
        pltpu.sync_copy(dis_hbm.at[sl], dbuf)
        b2v = b2buf[...]

        def fin(i, _):
            s = pl.ds(i * L, L)
            gg = g2t[pl.ds(sid * SLICE + i * L, L)]
            gbuf[s] = (gbuf[s] + gg) * dbuf[s] + b2v
            return 0
        lax.fori_loop(0, SLICE // L, fin, 0)
        pltpu.sync_copy(gbuf, out_hbm.at[sl])


def _agg2(g2, src, dst, dis, b2t):
    e = src.shape[0]
    ew = e // 16
    assert ew % 8 == 0 and ew % L == 0
    return pl.kernel(
        functools.partial(_agg2_body, ew),
        out_type=jax.ShapeDtypeStruct((NP,), jnp.float32),
        mesh=plsc.VectorSubcoreMesh(**_MESH),
        compiler_params=_SC_PARAMS_NOLAYOUT,
        scratch_types=[
            pltpu.VMEM_SHARED((NP,), jnp.float32),
            pltpu.VMEM((NP,), jnp.float32),
            pltpu.VMEM((ew,), jnp.int32),
            pltpu.VMEM((ew,), jnp.int32),
            pltpu.VMEM((ew,), jnp.float32),
            pltpu.VMEM((SLICE,), jnp.float32),
            pltpu.VMEM((SLICE,), jnp.float32),
            pltpu.VMEM((SLICE,), jnp.float32),
            pltpu.VMEM((L,), jnp.float32),
        ],
    )(g2, src, dst, dis, b2t)


def kernel(x, edge_index, W1, b1, W2, b2):
    src = edge_index[0]
    dst = edge_index[1]
    xp = jnp.pad(x, ((0, NP - N), (0, 0)))
    deg = _deg_hist(dst)
    g1, dis = _tc1(xp, W1, deg.reshape(NP, 1))
    p0, p1 = _agg1(g1, src, dst)
    g2 = _tc2(p0, p1, g1, dis, b1.reshape(1, H1), W2)
    b2t = jnp.tile(b2, L)
    outp = _agg2(g2.reshape(NP), src, dst, dis.reshape(NP), b2t)
    return outp[:N].reshape(N, 1)
